# baseline (speedup 1.0000x reference)
import functools

import jax
import jax.numpy as jnp
from jax.experimental import pallas as pl
from jax.experimental.pallas import tpu as pltpu

_BN = 4096


def _dcn_loss_kernel(emb_ref, cent_ref, out_ref, a_ref, *, inv_n):
    i = pl.program_id(0)
    ni = pl.num_programs(0)

    @pl.when(i == 0)
    def _build_a():
        c = cent_ref[...]
        c_sq = jnp.sum(c * c, axis=1, keepdims=True)
        a_f32 = jnp.concatenate([c, -0.5 * c_sq], axis=1)
        a_ref[...] = a_f32.T.astype(jnp.bfloat16)

    x = emb_ref[...]
    xb = jnp.concatenate(
        [x, jnp.ones((x.shape[0], 1), jnp.float32)], axis=1
    ).astype(jnp.bfloat16)

    g = jnp.dot(xb, a_ref[...], preferred_element_type=jnp.float32)
    part = jnp.max(g, axis=1, keepdims=True)

    x_sq = jnp.sum(x * x, axis=1, keepdims=True)
    s = jnp.sum(x_sq - 2.0 * part) * inv_n

    @pl.when(i == 0)
    def _first():
        out_ref[0, 0] = s

    @pl.when(i != 0)
    def _rest():
        out_ref[0, 0] = out_ref[0, 0] + s


def kernel(embedded, centers):
    n, d = embedded.shape
    k, _ = centers.shape
    ni = n // _BN

    total = pl.pallas_call(
        functools.partial(_dcn_loss_kernel, inv_n=1.0 / n),
        grid=(ni,),
        in_specs=[
            pl.BlockSpec((_BN, d), lambda i: (i, 0)),
            pl.BlockSpec((k, d), lambda i: (0, 0)),
        ],
        out_specs=pl.BlockSpec(memory_space=pltpu.SMEM),
        out_shape=jax.ShapeDtypeStruct((1, 1), jnp.float32),
        scratch_shapes=[pltpu.VMEM((d + 1, k), jnp.bfloat16)],
        compiler_params=pltpu.CompilerParams(
            dimension_semantics=("arbitrary",)
        ),
    )(embedded, centers)
    return total[0, 0]
